# baseline (speedup 1.0000x reference)
import jax
import jax.numpy as jnp
from jax.experimental import pallas as pl
from jax.experimental.pallas import tpu as pltpu

_B = 16384
_BS = 256
_NSUB = _B // _BS


def _seg_kernel(lo_c, hi_c,
                nodes_ref, ids_ref, glob_ref, w1_ref, w2_ref, b_ref,
                out_ref, accum_ref):
    t = pl.program_id(0)
    nsteps = pl.num_programs(0)

    @pl.when(t == 0)
    def _init():
        accum_ref[...] = jnp.full_like(accum_ref[...], -jnp.inf)

    def upd(g, bmax):
        cur = accum_ref[pl.ds(g, 1), :]
        accum_ref[pl.ds(g, 1), :] = jnp.maximum(cur, bmax)

    for j in range(_NSUB):
        sub_nodes = nodes_ref[j * _BS:(j + 1) * _BS, :]
        sub_ids = ids_ref[j * _BS:(j + 1) * _BS, :]
        idx = t * _NSUB + j
        lo = lo_c[idx]
        hi = hi_c[idx]

        @pl.when(lo == hi)
        def _fast(sub_nodes=sub_nodes, lo=lo):
            upd(lo, jnp.max(sub_nodes, axis=0, keepdims=True))

        @pl.when(lo != hi)
        def _slow(sub_nodes=sub_nodes, sub_ids=sub_ids, lo=lo, hi=hi):
            def body(g, _):
                vals = jnp.where(sub_ids == g, sub_nodes, -jnp.inf)
                upd(g, jnp.max(vals, axis=0, keepdims=True))
                return 0
            jax.lax.fori_loop(lo, hi + 1, body, 0)

    @pl.when(t == nsteps - 1)
    def _fin():
        gpad = accum_ref.shape[0]
        gidx = jax.lax.broadcasted_iota(jnp.int32, (gpad, 1), 0)
        nseg = glob_ref.shape[0]
        acc = jnp.where(gidx < nseg, accum_ref[...], 0.0)
        out = jnp.dot(acc, w1_ref[...], preferred_element_type=jnp.float32)
        out += jnp.dot(glob_ref[...], w2_ref[...],
                       preferred_element_type=jnp.float32)
        out_ref[...] = out + b_ref[...]


def kernel(nodes, segment_ids, globals_, W, b):
    n, d = nodes.shape
    g, dg = globals_.shape
    mlp = W.shape[1]
    gpad = 128
    nsteps = (n + _B - 1) // _B
    npad = nsteps * _B

    ids = segment_ids.astype(jnp.int32)
    ids_pad = jnp.full((npad,), g, jnp.int32).at[:n].set(ids)
    ids_2d = ids_pad.reshape(npad, 1)
    lo_c = ids_pad[::_BS]
    hi_c = ids_pad[_BS - 1::_BS]

    glob_pad = jnp.zeros((gpad, dg), jnp.float32).at[:g].set(globals_)
    w1 = W[:d]
    w2 = W[d:]
    b2 = b.reshape(1, mlp)

    grid_spec = pltpu.PrefetchScalarGridSpec(
        num_scalar_prefetch=2,
        grid=(nsteps,),
        in_specs=[
            pl.BlockSpec((_B, d), lambda t, lc, hc: (t, 0)),
            pl.BlockSpec((_B, 1), lambda t, lc, hc: (t, 0)),
            pl.BlockSpec((gpad, dg), lambda t, lc, hc: (0, 0)),
            pl.BlockSpec((d, mlp), lambda t, lc, hc: (0, 0)),
            pl.BlockSpec((dg, mlp), lambda t, lc, hc: (0, 0)),
            pl.BlockSpec((1, mlp), lambda t, lc, hc: (0, 0)),
        ],
        out_specs=pl.BlockSpec((gpad, mlp), lambda t, lc, hc: (0, 0)),
        scratch_shapes=[pltpu.VMEM((gpad, d), jnp.float32)],
    )

    out = pl.pallas_call(
        _seg_kernel,
        grid_spec=grid_spec,
        out_shape=jax.ShapeDtypeStruct((gpad, mlp), jnp.float32),
    )(lo_c, hi_c, nodes, ids_2d, glob_pad, w1, w2, b2)
    return out[:g]


# speedup vs baseline: 1.0249x; 1.0249x over previous
import jax
import jax.numpy as jnp
from jax.experimental import pallas as pl
from jax.experimental.pallas import tpu as pltpu

_B = 8192
_BS = 256
_NSUB = _B // _BS


def _seg_kernel(lo_c, hi_c,
                nodes_ref, ids_ref, glob_ref, w1_ref, w2_ref, b_ref,
                out_ref, accum_ref):
    t = pl.program_id(0)
    nsteps = pl.num_programs(0)

    @pl.when(t == 0)
    def _init():
        accum_ref[...] = jnp.full_like(accum_ref[...], -jnp.inf)

    def upd(g, bmax):
        cur = accum_ref[pl.ds(g, 1), :]
        accum_ref[pl.ds(g, 1), :] = jnp.maximum(cur, bmax)

    for j in range(_NSUB):
        sub_nodes = nodes_ref[j * _BS:(j + 1) * _BS, :]
        sub_ids = ids_ref[j * _BS:(j + 1) * _BS, :]
        idx = t * _NSUB + j
        lo = lo_c[idx]
        hi = hi_c[idx]

        @pl.when(lo == hi)
        def _fast(sub_nodes=sub_nodes, lo=lo):
            upd(lo, jnp.max(sub_nodes, axis=0, keepdims=True))

        @pl.when(lo != hi)
        def _slow(sub_nodes=sub_nodes, sub_ids=sub_ids, lo=lo, hi=hi):
            def body(g, _):
                vals = jnp.where(sub_ids == g, sub_nodes, -jnp.inf)
                upd(g, jnp.max(vals, axis=0, keepdims=True))
                return 0
            jax.lax.fori_loop(lo, hi + 1, body, 0)

    @pl.when(t == nsteps - 1)
    def _fin():
        gpad = accum_ref.shape[0]
        gidx = jax.lax.broadcasted_iota(jnp.int32, (gpad, 1), 0)
        nseg = glob_ref.shape[0]
        acc = jnp.where(gidx < nseg, accum_ref[...], 0.0)
        out = jnp.dot(acc, w1_ref[...], preferred_element_type=jnp.float32)
        out += jnp.dot(glob_ref[...], w2_ref[...],
                       preferred_element_type=jnp.float32)
        out_ref[...] = out + b_ref[...]


def kernel(nodes, segment_ids, globals_, W, b):
    n, d = nodes.shape
    g, dg = globals_.shape
    mlp = W.shape[1]
    gpad = 128
    nsteps = (n + _B - 1) // _B
    npad = nsteps * _B

    ids = segment_ids.astype(jnp.int32)
    ids_pad = jnp.full((npad,), g, jnp.int32).at[:n].set(ids)
    ids_2d = ids_pad.reshape(npad, 1)
    lo_c = ids_pad[::_BS]
    hi_c = ids_pad[_BS - 1::_BS]

    glob_pad = jnp.zeros((gpad, dg), jnp.float32).at[:g].set(globals_)
    w1 = W[:d]
    w2 = W[d:]
    b2 = b.reshape(1, mlp)

    grid_spec = pltpu.PrefetchScalarGridSpec(
        num_scalar_prefetch=2,
        grid=(nsteps,),
        in_specs=[
            pl.BlockSpec((_B, d), lambda t, lc, hc: (t, 0)),
            pl.BlockSpec((_B, 1), lambda t, lc, hc: (t, 0)),
            pl.BlockSpec((gpad, dg), lambda t, lc, hc: (0, 0)),
            pl.BlockSpec((d, mlp), lambda t, lc, hc: (0, 0)),
            pl.BlockSpec((dg, mlp), lambda t, lc, hc: (0, 0)),
            pl.BlockSpec((1, mlp), lambda t, lc, hc: (0, 0)),
        ],
        out_specs=pl.BlockSpec((gpad, mlp), lambda t, lc, hc: (0, 0)),
        scratch_shapes=[pltpu.VMEM((gpad, d), jnp.float32)],
    )

    out = pl.pallas_call(
        _seg_kernel,
        grid_spec=grid_spec,
        out_shape=jax.ShapeDtypeStruct((gpad, mlp), jnp.float32),
    )(lo_c, hi_c, nodes, ids_2d, glob_pad, w1, w2, b2)
    return out[:g]
